# baseline (speedup 1.0000x reference)
import functools

import jax
import jax.numpy as jnp
from jax import lax
from jax.experimental import pallas as pl
from jax.experimental.pallas import tpu as pltpu
from jax.experimental.pallas import tpu_sc as plsc

N = 1000000
DIM = 16
B = 16384

_NC = 2
_NS = 16
_NW = _NC * _NS
_BPW = B // _NW
_CHUNK = 128
_NCHUNK = _BPW // _CHUNK

_ROWS2D = B * DIM // 128
_PPR = 128 // DIM


def _sc_gather(left, right, emb, bias):
    mesh = plsc.VectorSubcoreMesh(core_axis_name="c", subcore_axis_name="s")

    @functools.partial(
        pl.kernel,
        out_type=(
            jax.ShapeDtypeStruct((B, DIM), jnp.float32),
            jax.ShapeDtypeStruct((B, DIM), jnp.float32),
            jax.ShapeDtypeStruct((B,), jnp.float32),
        ),
        mesh=mesh,
        scratch_types=[
            pltpu.VMEM((_NCHUNK, _CHUNK), jnp.int32),
            pltpu.VMEM((_NCHUNK, _CHUNK), jnp.int32),
            pltpu.VMEM((_BPW, DIM), jnp.float32),
            pltpu.VMEM((_BPW, DIM), jnp.float32),
            pltpu.VMEM((_BPW,), jnp.float32),
            pltpu.SemaphoreType.DMA,
        ],
        compiler_params=pltpu.CompilerParams(use_tc_tiling_on_sc=False),
    )
    def body(left_hbm, right_hbm, emb_hbm, bias_hbm,
             lrows_hbm, rrows_hbm, bvals_hbm,
             lidx, ridx, lrows, rrows, bvals, sem):
        wid = lax.axis_index("s") * _NC + lax.axis_index("c")
        base = wid * _BPW

        for c in range(_NCHUNK):
            pltpu.sync_copy(left_hbm.at[pl.ds(base + c * _CHUNK, _CHUNK)],
                            lidx.at[c])
            pltpu.sync_copy(right_hbm.at[pl.ds(base + c * _CHUNK, _CHUNK)],
                            ridx.at[c])

        handles = []
        for c in range(_NCHUNK):
            sl = pl.ds(c * _CHUNK, _CHUNK)
            handles.append(pltpu.async_copy(emb_hbm.at[lidx.at[c]],
                                            lrows.at[sl], sem))
            handles.append(pltpu.async_copy(emb_hbm.at[ridx.at[c]],
                                            rrows.at[sl], sem))
            handles.append(pltpu.async_copy(bias_hbm.at[ridx.at[c]],
                                            bvals.at[sl], sem))
        for h in handles:
            h.wait()

        pltpu.sync_copy(lrows, lrows_hbm.at[pl.ds(base, _BPW)])
        pltpu.sync_copy(rrows, rrows_hbm.at[pl.ds(base, _BPW)])
        pltpu.sync_copy(bvals, bvals_hbm.at[pl.ds(base, _BPW)])

    return body(left, right, emb, bias)


def _tc_loss_kernel(l_ref, r_ref, b_ref, y_ref, out_ref):
    prod = l_ref[...] * r_ref[...]
    seg = lax.broadcasted_iota(jnp.int32, (128, _PPR), 0) // DIM
    col = lax.broadcasted_iota(jnp.int32, (128, _PPR), 1)
    select = jnp.where(seg == col, 1.0, 0.0).astype(jnp.float32)
    score = jax.lax.dot_general(
        prod, select, (((1,), (0,)), ((), ())),
        preferred_element_type=jnp.float32)
    score = score + b_ref[...]
    prob = jax.nn.sigmoid(score)
    prob = jnp.clip(prob, 1e-05, 1 - 1e-05)
    y = y_ref[...]
    out_ref[0, 0] = -jnp.sum(y * jnp.log(prob) + (1 - y) * jnp.log(1 - prob))


def _tc_loss(lrows, rrows, bvals, y):
    out = pl.pallas_call(
        _tc_loss_kernel,
        out_shape=jax.ShapeDtypeStruct((1, 1), jnp.float32),
        out_specs=pl.BlockSpec(memory_space=pltpu.SMEM),
    )(lrows.reshape(_ROWS2D, 128), rrows.reshape(_ROWS2D, 128),
      bvals.reshape(B // _PPR, _PPR), y.reshape(B // _PPR, _PPR))
    return out[0, 0]


def kernel(left, right, y, emb, bias):
    lrows, rrows, bvals = _sc_gather(left.astype(jnp.int32),
                                     right.astype(jnp.int32), emb, bias)
    return _tc_loss(lrows, rrows, bvals, y)


# speedup vs baseline: 1.0034x; 1.0034x over previous
import functools

import jax
import jax.numpy as jnp
from jax import lax
from jax.experimental import pallas as pl
from jax.experimental.pallas import tpu as pltpu
from jax.experimental.pallas import tpu_sc as plsc

N = 1000000
DIM = 16
B = 16384

_NC = 2
_NS = 16
_NW = _NC * _NS
_BPW = B // _NW
_CHUNK = 128
_NCHUNK = _BPW // _CHUNK
_NGROUP = _BPW // 16


def _sc_scores(left, right, emb, bias):
    mesh = plsc.VectorSubcoreMesh(core_axis_name="c", subcore_axis_name="s")

    @functools.partial(
        pl.kernel,
        out_type=jax.ShapeDtypeStruct((B,), jnp.float32),
        mesh=mesh,
        scratch_types=[
            pltpu.VMEM((_NCHUNK, _CHUNK), jnp.int32),
            pltpu.VMEM((_NCHUNK, _CHUNK), jnp.int32),
            pltpu.VMEM((_BPW, DIM), jnp.float32),
            pltpu.VMEM((_BPW, DIM), jnp.float32),
            pltpu.VMEM((_BPW,), jnp.float32),
            pltpu.VMEM((_BPW,), jnp.float32),
            pltpu.SemaphoreType.DMA,
        ],
        compiler_params=pltpu.CompilerParams(use_tc_tiling_on_sc=False,
                                             needs_layout_passes=False),
    )
    def body(left_hbm, right_hbm, emb_hbm, bias_hbm, score_hbm,
             lidx, ridx, lrows, rrows, bvals, score_v, sem):
        wid = lax.axis_index("s") * _NC + lax.axis_index("c")
        base = wid * _BPW

        for c in range(_NCHUNK):
            pltpu.sync_copy(left_hbm.at[pl.ds(base + c * _CHUNK, _CHUNK)],
                            lidx.at[c])
            pltpu.sync_copy(right_hbm.at[pl.ds(base + c * _CHUNK, _CHUNK)],
                            ridx.at[c])

        handles = []
        for c in range(_NCHUNK):
            sl = pl.ds(c * _CHUNK, _CHUNK)
            handles.append(pltpu.async_copy(emb_hbm.at[lidx.at[c]],
                                            lrows.at[sl], sem))
            handles.append(pltpu.async_copy(emb_hbm.at[ridx.at[c]],
                                            rrows.at[sl], sem))
            handles.append(pltpu.async_copy(bias_hbm.at[ridx.at[c]],
                                            bvals.at[sl], sem))
        for h in handles:
            h.wait()

        iota16 = lax.iota(jnp.int32, 16)

        def group(g, carry):
            row0 = pl.multiple_of(g * 16, 16)
            rowidx = iota16 + row0
            acc = bvals[pl.ds(row0, 16)]
            for j in range(DIM):
                colidx = jnp.full((16,), j, jnp.int32)
                lv = plsc.load_gather(lrows, [rowidx, colidx])
                rv = plsc.load_gather(rrows, [rowidx, colidx])
                acc = acc + lv * rv
            score_v[pl.ds(row0, 16)] = acc
            return carry

        lax.fori_loop(0, _NGROUP, group, 0)
        pltpu.sync_copy(score_v, score_hbm.at[pl.ds(base, _BPW)])

    return body(left, right, emb, bias)


def _tc_loss_kernel(score_ref, y_ref, out_ref):
    s = score_ref[...]
    y = y_ref[...]
    prob = jax.nn.sigmoid(s)
    prob = jnp.clip(prob, 1e-05, 1 - 1e-05)
    out_ref[0, 0] = -jnp.sum(y * jnp.log(prob) + (1 - y) * jnp.log(1 - prob))


def _tc_loss(score, y):
    out = pl.pallas_call(
        _tc_loss_kernel,
        out_shape=jax.ShapeDtypeStruct((1, 1), jnp.float32),
        out_specs=pl.BlockSpec(memory_space=pltpu.SMEM),
    )(score.reshape(128, 128), y.reshape(128, 128))
    return out[0, 0]


def kernel(left, right, y, emb, bias):
    score = _sc_scores(left.astype(jnp.int32), right.astype(jnp.int32),
                       emb, bias)
    return _tc_loss(score, y)
